# baseline (speedup 1.0000x reference)
import jax
import jax.numpy as jnp
from jax.experimental import pallas as pl
from jax.experimental.pallas import tpu as pltpu

_HB = 256


def _head_kernel(x_ref, w_ref, b_ref, o_ref):
    X = x_ref[0]
    W = w_ref[...]
    o_ref[0] = (
        jax.lax.dot_general(W, X, (((1,), (0,)), ((), ())),
                            preferred_element_type=jnp.float32)
        + b_ref[...]
    )


def kernel(x, Wc, bc):
    B, C, H, W = x.shape
    out = pl.pallas_call(
        _head_kernel,
        grid=(B, H // _HB),
        in_specs=[
            pl.BlockSpec((1, C, _HB, W), lambda b, h: (b, 0, h, 0)),
            pl.BlockSpec((18, C), lambda b, h: (0, 0)),
            pl.BlockSpec((18, 1, 1), lambda b, h: (0, 0, 0)),
        ],
        out_specs=pl.BlockSpec((1, 18, _HB, W), lambda b, h: (b, 0, h, 0)),
        out_shape=jax.ShapeDtypeStruct((B, 18, H, W), jnp.float32),
        compiler_params=pltpu.CompilerParams(
            dimension_semantics=("parallel", "parallel")),
    )(x, Wc, bc.reshape(18, 1, 1))
    return jnp.transpose(out.reshape(B, 3, 6, H, W), (0, 1, 3, 4, 2))
